# baseline (speedup 1.0000x reference)
import functools

import jax
import jax.numpy as jnp
from jax import lax
from jax.experimental import pallas as pl
from jax.experimental.pallas import tpu as pltpu
from jax.experimental.pallas import tpu_sc as plsc

D = 64
B_ROWS = 4096
SEQ = 200
NC = 2
NS = 16
NW = NC * NS
RPW = B_ROWS // NW
C0 = 128
C1 = SEQ - C0
NB = 4
LA = 2

_mesh = plsc.VectorSubcoreMesh(core_axis_name="c", subcore_axis_name="s")


@functools.partial(
    pl.kernel,
    out_type=jax.ShapeDtypeStruct((B_ROWS, SEQ, D), jnp.float32),
    mesh=_mesh,
    scratch_types=[
        pltpu.VMEM((RPW, SEQ), jnp.int32),
        [pltpu.VMEM((SEQ, D), jnp.float32)] * NB,
        [pltpu.SemaphoreType.DMA] * NB,
        [pltpu.SemaphoreType.DMA] * NB,
    ],
    compiler_params=pltpu.CompilerParams(use_tc_tiling_on_sc=False),
)
def _gather_kernel(idx_hbm, table_hbm, out_hbm, idx_v, rows, sem_g, sem_o):
    wid = lax.axis_index("s") * NC + lax.axis_index("c")
    row0 = wid * RPW
    pltpu.sync_copy(idx_hbm.at[pl.ds(row0, RPW)], idx_v)

    def fire_gathers(r, b):
        pltpu.async_copy(table_hbm.at[idx_v.at[r, pl.ds(0, C0)]],
                         rows[b].at[pl.ds(0, C0)], sem_g[b])
        pltpu.async_copy(table_hbm.at[idx_v.at[r, pl.ds(C0, C1)]],
                         rows[b].at[pl.ds(C0, C1)], sem_g[b])

    def wait_gathers(r, b):
        pltpu.make_async_copy(table_hbm.at[idx_v.at[r, pl.ds(0, C0)]],
                              rows[b].at[pl.ds(0, C0)], sem_g[b]).wait()
        pltpu.make_async_copy(table_hbm.at[idx_v.at[r, pl.ds(C0, C1)]],
                              rows[b].at[pl.ds(C0, C1)], sem_g[b]).wait()

    def fire_out(r, b):
        pltpu.async_copy(rows[b], out_hbm.at[row0 + r], sem_o[b])

    def wait_out(r, b):
        pltpu.make_async_copy(rows[b], out_hbm.at[row0 + r], sem_o[b]).wait()

    for b in range(LA):
        fire_gathers(b, b)

    for b in range(NB):
        r = b
        wait_gathers(r, b)
        fire_out(r, b)
        bn = (b + LA) % NB
        if r >= LA:
            wait_out(r - LA, bn)
        fire_gathers(r + LA, bn)

    def round_body(g, carry):
        r0 = g * NB
        for b in range(NB):
            r = r0 + b
            wait_gathers(r, b)
            fire_out(r, b)
            bn = (b + LA) % NB
            wait_out(r - LA, bn)
            fire_gathers(r + LA, bn)
        return carry

    lax.fori_loop(1, RPW // NB - 1, round_body, 0)

    r0 = RPW - NB
    for b in range(NB):
        r = r0 + b
        wait_gathers(r, b)
        fire_out(r, b)
        if b < LA:
            bn = (b + LA) % NB
            wait_out(r - LA, bn)
            fire_gathers(r + LA, bn)

    for b in range(NB):
        wait_out(r0 + b, b)


def kernel(input_, weight):
    return _gather_kernel(input_.astype(jnp.int32), weight)


# speedup vs baseline: 1.0006x; 1.0006x over previous
import functools

import jax
import jax.numpy as jnp
from jax import lax
from jax.experimental import pallas as pl
from jax.experimental.pallas import tpu as pltpu
from jax.experimental.pallas import tpu_sc as plsc

D = 64
B_ROWS = 4096
SEQ = 200
NC = 2
NS = 16
NW = NC * NS
RPW = B_ROWS // NW
C0 = 128
C1 = SEQ - C0
NB = 8
LA = 4

_mesh = plsc.VectorSubcoreMesh(core_axis_name="c", subcore_axis_name="s")


@functools.partial(
    pl.kernel,
    out_type=jax.ShapeDtypeStruct((B_ROWS, SEQ, D), jnp.float32),
    mesh=_mesh,
    scratch_types=[
        pltpu.VMEM((RPW, SEQ), jnp.int32),
        [pltpu.VMEM((SEQ, D), jnp.float32)] * NB,
        [pltpu.SemaphoreType.DMA] * NB,
        [pltpu.SemaphoreType.DMA] * NB,
    ],
    compiler_params=pltpu.CompilerParams(use_tc_tiling_on_sc=False),
)
def _gather_kernel(idx_hbm, table_hbm, out_hbm, idx_v, rows, sem_g, sem_o):
    wid = lax.axis_index("s") * NC + lax.axis_index("c")
    row0 = wid * RPW
    pltpu.sync_copy(idx_hbm.at[pl.ds(row0, RPW)], idx_v)

    def fire_gathers(r, b):
        pltpu.async_copy(table_hbm.at[idx_v.at[r, pl.ds(0, C0)]],
                         rows[b].at[pl.ds(0, C0)], sem_g[b])
        pltpu.async_copy(table_hbm.at[idx_v.at[r, pl.ds(C0, C1)]],
                         rows[b].at[pl.ds(C0, C1)], sem_g[b])

    def wait_gathers(r, b):
        pltpu.make_async_copy(table_hbm.at[idx_v.at[r, pl.ds(0, C0)]],
                              rows[b].at[pl.ds(0, C0)], sem_g[b]).wait()
        pltpu.make_async_copy(table_hbm.at[idx_v.at[r, pl.ds(C0, C1)]],
                              rows[b].at[pl.ds(C0, C1)], sem_g[b]).wait()

    def fire_out(r, b):
        pltpu.async_copy(rows[b], out_hbm.at[row0 + r], sem_o[b])

    def wait_out(r, b):
        pltpu.make_async_copy(rows[b], out_hbm.at[row0 + r], sem_o[b]).wait()

    for b in range(LA):
        fire_gathers(b, b)

    for b in range(NB):
        r = b
        wait_gathers(r, b)
        fire_out(r, b)
        bn = (b + LA) % NB
        if r >= LA:
            wait_out(r - LA, bn)
        fire_gathers(r + LA, bn)

    def round_body(g, carry):
        r0 = g * NB
        for b in range(NB):
            r = r0 + b
            wait_gathers(r, b)
            fire_out(r, b)
            bn = (b + LA) % NB
            wait_out(r - LA, bn)
            fire_gathers(r + LA, bn)
        return carry

    lax.fori_loop(1, RPW // NB - 1, round_body, 0)

    r0 = RPW - NB
    for b in range(NB):
        r = r0 + b
        wait_gathers(r, b)
        fire_out(r, b)
        if b < LA:
            bn = (b + LA) % NB
            wait_out(r - LA, bn)
            fire_gathers(r + LA, bn)

    for b in range(NB):
        wait_out(r0 + b, b)


def kernel(input_, weight):
    return _gather_kernel(input_.astype(jnp.int32), weight)


# speedup vs baseline: 1.0016x; 1.0010x over previous
import functools

import jax
import jax.numpy as jnp
from jax import lax
from jax.experimental import pallas as pl
from jax.experimental.pallas import tpu as pltpu
from jax.experimental.pallas import tpu_sc as plsc

D = 64
B_ROWS = 4096
SEQ = 200
NC = 2
NS = 16
NW = NC * NS
RPW = B_ROWS // NW
C0 = 128
C1 = SEQ - C0
NB = 4
LA = 2

_mesh = plsc.VectorSubcoreMesh(core_axis_name="c", subcore_axis_name="s")


@functools.partial(
    pl.kernel,
    out_type=jax.ShapeDtypeStruct((B_ROWS, SEQ, D), jnp.float32),
    mesh=_mesh,
    scratch_types=[
        pltpu.VMEM((RPW, SEQ), jnp.int32),
        [pltpu.VMEM((SEQ, D), jnp.float32)] * NB,
        [pltpu.SemaphoreType.DMA] * NB,
        [pltpu.SemaphoreType.DMA] * NB,
    ],
    compiler_params=pltpu.CompilerParams(use_tc_tiling_on_sc=False),
)
def _gather_kernel(idx_hbm, table_hbm, out_hbm, idx_v, rows, sem_g, sem_o):
    wid = lax.axis_index("s") * NC + lax.axis_index("c")
    row0 = wid * RPW
    pltpu.sync_copy(idx_hbm.at[pl.ds(row0, RPW)], idx_v)

    def fire_gathers(r, b):
        pltpu.async_copy(table_hbm.at[idx_v.at[r, pl.ds(0, C0)]],
                         rows[b].at[pl.ds(0, C0)], sem_g[b])
        pltpu.async_copy(table_hbm.at[idx_v.at[r, pl.ds(C0, C1)]],
                         rows[b].at[pl.ds(C0, C1)], sem_g[b])

    def wait_gathers(r, b):
        pltpu.make_async_copy(table_hbm.at[idx_v.at[r, pl.ds(0, C0)]],
                              rows[b].at[pl.ds(0, C0)], sem_g[b]).wait()
        pltpu.make_async_copy(table_hbm.at[idx_v.at[r, pl.ds(C0, C1)]],
                              rows[b].at[pl.ds(C0, C1)], sem_g[b]).wait()

    def fire_out(r, b):
        pltpu.async_copy(rows[b], out_hbm.at[row0 + r], sem_o[b])

    def wait_out(r, b):
        pltpu.make_async_copy(rows[b], out_hbm.at[row0 + r], sem_o[b]).wait()

    for b in range(LA):
        fire_gathers(b, b)

    for b in range(NB):
        r = b
        wait_gathers(r, b)
        fire_out(r, b)
        bn = (b + LA) % NB
        if r >= LA:
            wait_out(r - LA, bn)
        fire_gathers(r + LA, bn)

    def round_body(g, carry):
        r0 = g * NB
        for b in range(NB):
            r = r0 + b
            wait_gathers(r, b)
            fire_out(r, b)
            bn = (b + LA) % NB
            wait_out(r - LA, bn)
            fire_gathers(r + LA, bn)
        return carry

    lax.fori_loop(1, RPW // NB - 1, round_body, 0)

    r0 = RPW - NB
    for b in range(NB):
        r = r0 + b
        wait_gathers(r, b)
        fire_out(r, b)
        if b < LA:
            bn = (b + LA) % NB
            wait_out(r - LA, bn)
            fire_gathers(r + LA, bn)

    for b in range(NB):
        wait_out(r0 + b, b)


def kernel(input_, weight):
    return _gather_kernel(input_.astype(jnp.int32), weight)
